# baseline (speedup 1.0000x reference)
import functools

import jax
import jax.numpy as jnp
from jax import lax
from jax.experimental import pallas as pl
from jax.experimental.pallas import tpu as pltpu
from jax.experimental.pallas import tpu_sc as plsc

BATCH = 16384
LATENT = 64
USERS = 1000000
MOVIES = 100000
PAIR = 2 * LATENT
NC = 2
NS = 16
NW = NC * NS
BPW = BATCH // NW
LANES = 16
ICHUNK = 128
NIC = BPW // ICHUNK
NBLK = ICHUNK // LANES


def _make_kernel():
    mesh = plsc.VectorSubcoreMesh(
        core_axis_name="c", subcore_axis_name="s", num_cores=NC, num_subcores=NS
    )

    slab = pltpu.VMEM((ICHUNK, PAIR), jnp.float32)

    @functools.partial(
        pl.kernel,
        out_type=jax.ShapeDtypeStruct((BATCH,), jnp.float32),
        mesh=mesh,
        scratch_types=[
            pltpu.VMEM((NIC, ICHUNK), jnp.int32),
            pltpu.VMEM((NIC, ICHUNK), jnp.int32),
            pltpu.VMEM((NIC, ICHUNK), jnp.int32),
            pltpu.VMEM((NIC, ICHUNK), jnp.int32),
            slab, slab,
            slab, slab,
            pltpu.VMEM((BPW,), jnp.float32),
            pltpu.SemaphoreType.DMA,
            pltpu.SemaphoreType.DMA,
            pltpu.SemaphoreType.DMA,
            pltpu.SemaphoreType.DMA,
        ],
        compiler_params=pltpu.CompilerParams(needs_layout_passes=False),
    )
    def svd_dot(u_hbm, m_hbm, ut_hbm, mt_hbm, out_hbm,
                uidx, midx, ugrp, mgrp, ubuf0, ubuf1, mbuf0, mbuf1, outv,
                sem_u0, sem_u1, sem_m0, sem_m1):
        wid = lax.axis_index("s") * NC + lax.axis_index("c")
        base = wid * BPW

        pltpu.sync_copy(u_hbm.at[wid], uidx)
        pltpu.sync_copy(m_hbm.at[wid], midx)

        for c in range(NIC):
            for j in range(NBLK):
                sl = pl.ds(j * LANES, LANES)
                ugrp[c, sl] = lax.shift_right_logical(uidx[c, sl], 1)
                mgrp[c, sl] = lax.shift_right_logical(midx[c, sl], 1)

        ubuf = (ubuf0, ubuf1)
        mbuf = (mbuf0, mbuf1)
        sem_u = (sem_u0, sem_u1)
        sem_m = (sem_m0, sem_m1)

        def fire(c):
            b = c % 2
            return (
                pltpu.async_copy(ut_hbm.at[ugrp.at[c]], ubuf[b], sem_u[b]),
                pltpu.async_copy(mt_hbm.at[mgrp.at[c]], mbuf[b], sem_m[b]),
            )

        item0 = lax.iota(jnp.int32, LANES)

        def compute(c):
            b = c % 2
            ub, mb = ubuf[b], mbuf[b]

            def block(j, carry):
                sl = pl.ds(j * LANES, LANES)
                item = item0 + j * LANES
                ucol0 = jnp.bitwise_and(uidx[c, sl], 1) * LATENT
                mcol0 = jnp.bitwise_and(midx[c, sl], 1) * LATENT
                acc = jnp.zeros((LANES,), jnp.float32)
                for d in range(LATENT):
                    uv = plsc.load_gather(ub, [item, ucol0 + d])
                    mv = plsc.load_gather(mb, [item, mcol0 + d])
                    acc = acc + uv * mv
                outv[pl.ds(c * ICHUNK + j * LANES, LANES)] = acc
                return carry

            lax.fori_loop(0, NBLK, block, 0)

        handles = [None] * NIC
        handles[0] = fire(0)
        handles[1] = fire(1)
        for c in range(NIC):
            hu, hm = handles[c]
            hu.wait()
            hm.wait()
            compute(c)
            if c + 2 < NIC:
                handles[c + 2] = fire(c + 2)

        pltpu.sync_copy(outv, out_hbm.at[pl.ds(base, BPW)])

    return svd_dot


_svd_dot = _make_kernel()


@jax.jit
def kernel(u, m, user_table, movie_table):
    out = _svd_dot(u.astype(jnp.int32).reshape(NW, NIC, ICHUNK),
                   m.astype(jnp.int32).reshape(NW, NIC, ICHUNK),
                   user_table.reshape(USERS // 2, PAIR),
                   movie_table.reshape(MOVIES // 2, PAIR))
    return out.reshape(BATCH, 1)
